# baseline (speedup 1.0000x reference)
import functools

import jax
import jax.numpy as jnp
from jax import lax
from jax.experimental import pallas as pl
from jax.experimental.pallas import tpu as pltpu
from jax.experimental.pallas import tpu_sc as plsc

F32 = jnp.float32
BF16 = jnp.bfloat16
I32 = jnp.int32
NW = 32
KP1 = 17


def _ceil16(x):
    return (x + 15) // 16 * 16


def _ceil32(x):
    return (x + 31) // 32 * 32


def _mm3_body(x_ref, w_ref, o_ref):
    o_ref[0] = lax.dot(x_ref[...], w_ref[0],
                       precision=lax.Precision.HIGHEST,
                       preferred_element_type=F32).astype(o_ref.dtype)


def _tc_matmul_slots(x, w3, bm=512):
    m, p = x.shape
    nk, _, dpad = w3.shape
    return pl.pallas_call(
        _mm3_body,
        grid=(m // bm, nk),
        in_specs=[pl.BlockSpec((bm, p), lambda i, k: (i, 0)),
                  pl.BlockSpec((1, p, dpad), lambda i, k: (k, 0, 0))],
        out_specs=pl.BlockSpec((1, bm, dpad), lambda i, k: (k, i, 0)),
        out_shape=jax.ShapeDtypeStruct((nk, m, dpad), BF16),
    )(x, w3)


def _seg_list(total):
    segs, off = [], 0
    while off < total:
        seg = min(128, total - off)
        segs.append((off, seg))
        off += seg
    return segs


def _sc_gather_sum(ytab, idx, bias, nn, dpad, c):
    npw = nn // NW
    nchunks = npw // c
    segs = _seg_list(c * KP1)
    ngroups = dpad // 32
    mesh = plsc.VectorSubcoreMesh(core_axis_name="c", subcore_axis_name="s")

    @functools.partial(
        pl.kernel, mesh=mesh,
        out_type=jax.ShapeDtypeStruct((nn, dpad), F32),
        compiler_params=pltpu.CompilerParams(use_tc_tiling_on_sc=False,
                                             needs_layout_passes=False),
        scratch_types=[
            pltpu.VMEM((c * KP1,), I32), pltpu.VMEM((c * KP1,), I32),
            pltpu.VMEM((c * KP1, dpad), BF16),
            pltpu.VMEM((c * KP1, dpad), BF16),
            pltpu.VMEM((c, dpad), F32),
            pltpu.VMEM((dpad,), F32),
            pltpu.SemaphoreType.DMA, pltpu.SemaphoreType.DMA,
        ],
    )
    def k(y_hbm, idx_hbm, bias_hbm, out_hbm, idx_a, idx_b, rows_a, rows_b,
          out_v, bias_v, sem_a, sem_b):
        wid = lax.axis_index("s") * 2 + lax.axis_index("c")
        base_node = wid * npw
        pltpu.sync_copy(bias_hbm, bias_v)
        idx_bufs, row_bufs, sems = (idx_a, idx_b), (rows_a, rows_b), \
            (sem_a, sem_b)

        def fire(g, bi):
            node0 = base_node + g * c
            pltpu.sync_copy(idx_hbm.at[pl.ds(node0 * KP1, c * KP1)],
                            idx_bufs[bi])
            for off, seg in segs:
                pltpu.async_copy(y_hbm.at[idx_bufs[bi].at[pl.ds(off, seg)]],
                                 row_bufs[bi].at[pl.ds(off, seg)], sems[bi])

        def drain(bi):
            for off, seg in segs:
                pltpu.make_async_copy(
                    y_hbm.at[idx_bufs[bi].at[pl.ds(off, seg)]],
                    row_bufs[bi].at[pl.ds(off, seg)], sems[bi]).wait()

        def process(g, bi):
            rows_v = row_bufs[bi]

            def acc_body(i, carry2):
                r0 = i * KP1
                for grp in range(ngroups):
                    a32 = rows_v[r0, pl.ds(32 * grp, 32)]
                    for kk in range(1, KP1):
                        a32 = a32 + rows_v[r0 + kk, pl.ds(32 * grp, 32)]
                    aa, ab = plsc.unpack(a32,
                                         format=plsc.PackFormat.INTERLEAVED)
                    sla = pl.ds(32 * grp, 16)
                    slb = pl.ds(32 * grp + 16, 16)
                    out_v[i, sla] = jnp.maximum(aa + bias_v[sla], 0.0)
                    out_v[i, slb] = jnp.maximum(ab + bias_v[slb], 0.0)
                return carry2

            lax.fori_loop(0, c, acc_body, 0)
            pltpu.sync_copy(out_v, out_hbm.at[pl.ds(base_node + g * c, c)])

        fire(0, 0)

        def group(to, carry):
            for bb in (0, 1):
                g = 2 * to + bb

                @pl.when(g + 1 < nchunks)
                def _():
                    fire(g + 1, 1 - bb)

                drain(bb)
                process(g, bb)
            return carry

        lax.fori_loop(0, nchunks // 2, group, 0)

    return k(ytab, idx, bias)


def _sc_masked_mean(z, n_arr, b, e, dpad, fdim):
    fpad = _ceil16(fdim)
    nsl = fpad // 16
    rows_chunk = 512
    nch = e // rows_chunk
    mesh = plsc.VectorSubcoreMesh(core_axis_name="c", subcore_axis_name="s")

    @functools.partial(
        pl.kernel, mesh=mesh,
        out_type=jax.ShapeDtypeStruct((b, fpad), F32),
        compiler_params=pltpu.CompilerParams(use_tc_tiling_on_sc=False),
        scratch_types=[
            pltpu.VMEM((rows_chunk, dpad), F32),
            pltpu.VMEM((16,), I32),
            pltpu.VMEM((fpad,), F32),
        ],
    )
    def k(z_hbm, n_hbm, out_hbm, zrows_v, n_v, out_v):
        wid = lax.axis_index("s") * 2 + lax.axis_index("c")

        @pl.when(wid < b)
        def _():
            _masked_mean_worker(z_hbm, n_hbm, out_hbm, zrows_v, n_v, out_v,
                                wid, e, dpad, fdim, nsl, rows_chunk, nch)

    return k(z, n_arr)


def _masked_mean_worker(z_hbm, n_hbm, out_hbm, zrows_v, n_v, out_v, wid, e,
                        dpad, fdim, nsl, rows_chunk, nch):
        pltpu.sync_copy(n_hbm.at[wid], n_v)
        nsplat = n_v[pl.ds(0, 16)]
        iota = lax.iota(I32, 16)
        accs = [jnp.zeros((16,), F32) for _ in range(nsl)]
        for ch in range(nch):
            pltpu.sync_copy(z_hbm.at[pl.ds(wid * e + ch * rows_chunk,
                                           rows_chunk)], zrows_v)

            def ebody(i, carry):
                pred = (ch * rows_chunk + i) < nsplat
                out = []
                for s in range(nsl):
                    lanes_valid = 16 * s + iota < fdim
                    v = jnp.where(pred & lanes_valid,
                                  zrows_v[i, pl.ds(16 * s, 16)], 0.0)
                    out.append(carry[s] + v)
                return tuple(out)

            accs = lax.fori_loop(0, rows_chunk, ebody, tuple(accs))
        inv = 1.0 / jnp.maximum(nsplat, 1).astype(F32)
        for s in range(nsl):
            out_v[pl.ds(16 * s, 16)] = accs[s] * inv
        pltpu.sync_copy(out_v, out_hbm.at[wid])


def _head_body(x_ref, w1_ref, b1_ref, w2_ref, b2_ref, w3_ref, b3_ref,
               lg_ref, pred_ref):
    x = x_ref[...]
    h = jnp.maximum(lax.dot(x, w1_ref[...], precision=lax.Precision.HIGHEST,
                            preferred_element_type=F32) + b1_ref[...], 0.0)
    h = jnp.maximum(lax.dot(h, w2_ref[...], precision=lax.Precision.HIGHEST,
                            preferred_element_type=F32) + b2_ref[...], 0.0)
    lg = lax.dot(h, w3_ref[...], precision=lax.Precision.HIGHEST,
                 preferred_element_type=F32) + b3_ref[...]
    lg_ref[...] = lg
    ncls = lg.shape[1]
    col = lax.broadcasted_iota(I32, lg.shape, 1)
    mx = jnp.max(lg, axis=1, keepdims=True)
    pred_ref[...] = jnp.min(jnp.where(lg >= mx, col, ncls), axis=1,
                            keepdims=True)


def _tc_head(flat, w1, b1, w2, b2, w3, b3):
    b = flat.shape[0]
    ncls = w3.shape[1]
    return pl.pallas_call(
        _head_body,
        out_shape=(jax.ShapeDtypeStruct((b, ncls), F32),
                   jax.ShapeDtypeStruct((b, 1), I32)),
    )(flat, w1, b1, w2, b2, w3, b3)


def _build_wbig(wa, ws, fa, fs, p, out, dpad):
    kk = wa.shape[0] // (fa + fs)
    wa_r = wa.reshape(kk, fa + fs, out)
    ws_r = ws.reshape(kk, fa + fs, out)
    blocks = jnp.concatenate([wa_r, ws_r], axis=2)
    corr = -jnp.concatenate([wa_r[:, fa:, :].sum(0),
                             ws_r[:, fa:, :].sum(0)], axis=1)
    corr_full = jnp.zeros((fa + fs, 2 * out), F32).at[fa:].set(corr)
    wb = jnp.concatenate([blocks, corr_full[None]], axis=0)
    wb = jnp.pad(wb, ((0, 0), (0, p - (fa + fs)), (0, dpad - 2 * out)))
    perm = []
    for s in range(dpad // 32):
        for t in range(16):
            perm.extend((32 * s + t, 32 * s + 16 + t))
    wb = wb[:, :, jnp.array(perm, dtype=I32)]
    return wb


def kernel(space_features, all_features, neighbors_matrix, num_entries,
           params):
    b, e, fs0 = space_features.shape
    fa0 = all_features.shape[2]
    kk = neighbors_matrix.shape[2]
    nn = b * e
    nlayers = 6
    layer_out = [params['W%da' % l].shape[1] for l in range(nlayers)]

    nbr = neighbors_matrix.astype(I32)
    bofs = (jnp.arange(b, dtype=I32) * e)[:, None, None]
    gnbr = bofs + nbr
    idx_nbr = jnp.arange(kk, dtype=I32)[None, None, :] * nn + gnbr
    self_row = (kk * nn + bofs[..., 0]
                + jnp.arange(e, dtype=I32)[None, :])[:, :, None]
    idx = jnp.concatenate([idx_nbr, self_row], axis=2).reshape(-1)

    z = jnp.concatenate([all_features.reshape(nn, fa0),
                         space_features.reshape(nn, fs0)], axis=1)
    fa, fs = fa0, fs0
    for l in range(nlayers):
        out = layer_out[l]
        dpad = _ceil32(2 * out)
        p = z.shape[1]
        wb3 = _build_wbig(params['W%da' % l], params['W%ds' % l],
                          fa, fs, p, out, dpad)
        bias = jnp.pad(jnp.concatenate([params['b%da' % l],
                                        params['b%ds' % l]]),
                       (0, dpad - 2 * out))
        y3 = _tc_matmul_slots(z, wb3)
        ytab = y3.reshape(KP1 * nn, dpad)
        c = 128 if dpad <= 32 else 64
        z = _sc_gather_sum(ytab, idx, bias, nn, dpad, c)
        fa = fs = out

    n_rep = jnp.tile(num_entries.reshape(b, 1).astype(I32), (1, 16))
    flat = _sc_masked_mean(z, n_rep, b, e, z.shape[1], layer_out[-1])
    f1 = jnp.pad(params['fc1_w'], ((0, flat.shape[1] - layer_out[-1]),
                                   (0, 0)))
    logits, pred = _tc_head(flat, f1, params['fc1_b'][None],
                            params['fc2_w'], params['fc2_b'][None],
                            params['fc3_w'], params['fc3_b'][None])
    return logits, pred[:, 0]


# speedup vs baseline: 1.6872x; 1.6872x over previous
import functools

import jax
import jax.numpy as jnp
from jax import lax
from jax.experimental import pallas as pl
from jax.experimental.pallas import tpu as pltpu
from jax.experimental.pallas import tpu_sc as plsc

F32 = jnp.float32
BF16 = jnp.bfloat16
I32 = jnp.int32
NW = 32
KP1 = 17


def _ceil16(x):
    return (x + 15) // 16 * 16


def _ceil32(x):
    return (x + 31) // 32 * 32


def _mm3_body(x_ref, w_ref, o_ref):
    x = x_ref[...]
    for k in range(o_ref.shape[0]):
        o_ref[k] = lax.dot(x, w_ref[k],
                           precision=lax.Precision.HIGHEST,
                           preferred_element_type=F32).astype(o_ref.dtype)


def _tc_matmul_slots(x, w3, bm=512):
    m, p = x.shape
    nk, _, dpad = w3.shape
    return pl.pallas_call(
        _mm3_body,
        grid=(m // bm,),
        in_specs=[pl.BlockSpec((bm, p), lambda i: (i, 0)),
                  pl.BlockSpec((nk, p, dpad), lambda i: (0, 0, 0))],
        out_specs=pl.BlockSpec((nk, bm, dpad), lambda i: (0, i, 0)),
        out_shape=jax.ShapeDtypeStruct((nk, m, dpad), BF16),
    )(x, w3)


def _seg_list(total):
    segs, off = [], 0
    while off < total:
        seg = min(128, total - off)
        segs.append((off, seg))
        off += seg
    return segs


def _sc_gather_sum(ytab, idx, bias, nn, dpad, c):
    npw = nn // NW
    nchunks = npw // c
    segs = _seg_list(c * KP1)
    ngroups = dpad // 32
    mesh = plsc.VectorSubcoreMesh(core_axis_name="c", subcore_axis_name="s")

    @functools.partial(
        pl.kernel, mesh=mesh,
        out_type=jax.ShapeDtypeStruct((nn, dpad), F32),
        compiler_params=pltpu.CompilerParams(use_tc_tiling_on_sc=False,
                                             needs_layout_passes=False),
        scratch_types=[
            pltpu.VMEM((c * KP1,), I32), pltpu.VMEM((c * KP1,), I32),
            pltpu.VMEM((c * KP1, dpad), BF16),
            pltpu.VMEM((c * KP1, dpad), BF16),
            pltpu.VMEM((c, dpad), F32),
            pltpu.VMEM((dpad,), F32),
            pltpu.SemaphoreType.DMA, pltpu.SemaphoreType.DMA,
        ],
    )
    def k(y_hbm, idx_hbm, bias_hbm, out_hbm, idx_a, idx_b, rows_a, rows_b,
          out_v, bias_v, sem_a, sem_b):
        wid = lax.axis_index("s") * 2 + lax.axis_index("c")
        base_node = wid * npw
        pltpu.sync_copy(bias_hbm, bias_v)
        idx_bufs, row_bufs, sems = (idx_a, idx_b), (rows_a, rows_b), \
            (sem_a, sem_b)

        def fire(g, bi):
            node0 = base_node + g * c
            pltpu.sync_copy(idx_hbm.at[pl.ds(node0 * KP1, c * KP1)],
                            idx_bufs[bi])
            for off, seg in segs:
                pltpu.async_copy(y_hbm.at[idx_bufs[bi].at[pl.ds(off, seg)]],
                                 row_bufs[bi].at[pl.ds(off, seg)], sems[bi])

        def drain(bi):
            for off, seg in segs:
                pltpu.make_async_copy(
                    y_hbm.at[idx_bufs[bi].at[pl.ds(off, seg)]],
                    row_bufs[bi].at[pl.ds(off, seg)], sems[bi]).wait()

        def process(g, bi):
            rows_v = row_bufs[bi]

            def acc_body(i, carry2):
                r0 = i * KP1
                for grp in range(ngroups):
                    a32 = rows_v[r0, pl.ds(32 * grp, 32)]
                    for kk in range(1, KP1):
                        a32 = a32 + rows_v[r0 + kk, pl.ds(32 * grp, 32)]
                    aa, ab = plsc.unpack(a32,
                                         format=plsc.PackFormat.INTERLEAVED)
                    sla = pl.ds(32 * grp, 16)
                    slb = pl.ds(32 * grp + 16, 16)
                    out_v[i, sla] = jnp.maximum(aa + bias_v[sla], 0.0)
                    out_v[i, slb] = jnp.maximum(ab + bias_v[slb], 0.0)
                return carry2

            lax.fori_loop(0, c, acc_body, 0)
            pltpu.sync_copy(out_v, out_hbm.at[pl.ds(base_node + g * c, c)])

        fire(0, 0)

        def group(to, carry):
            for bb in (0, 1):
                g = 2 * to + bb

                @pl.when(g + 1 < nchunks)
                def _():
                    fire(g + 1, 1 - bb)

                drain(bb)
                process(g, bb)
            return carry

        lax.fori_loop(0, nchunks // 2, group, 0)

    return k(ytab, idx, bias)


def _sc_masked_mean(z, n_arr, b, e, dpad, fdim):
    fpad = _ceil16(fdim)
    nsl = fpad // 16
    rows_chunk = 512
    nch = e // rows_chunk
    mesh = plsc.VectorSubcoreMesh(core_axis_name="c", subcore_axis_name="s")

    @functools.partial(
        pl.kernel, mesh=mesh,
        out_type=jax.ShapeDtypeStruct((b, fpad), F32),
        compiler_params=pltpu.CompilerParams(use_tc_tiling_on_sc=False),
        scratch_types=[
            pltpu.VMEM((rows_chunk, dpad), F32),
            pltpu.VMEM((16,), I32),
            pltpu.VMEM((fpad,), F32),
        ],
    )
    def k(z_hbm, n_hbm, out_hbm, zrows_v, n_v, out_v):
        wid = lax.axis_index("s") * 2 + lax.axis_index("c")

        @pl.when(wid < b)
        def _():
            _masked_mean_worker(z_hbm, n_hbm, out_hbm, zrows_v, n_v, out_v,
                                wid, e, dpad, fdim, nsl, rows_chunk, nch)

    return k(z, n_arr)


def _masked_mean_worker(z_hbm, n_hbm, out_hbm, zrows_v, n_v, out_v, wid, e,
                        dpad, fdim, nsl, rows_chunk, nch):
        pltpu.sync_copy(n_hbm.at[wid], n_v)
        nsplat = n_v[pl.ds(0, 16)]
        iota = lax.iota(I32, 16)
        accs = [jnp.zeros((16,), F32) for _ in range(nsl)]
        for ch in range(nch):
            pltpu.sync_copy(z_hbm.at[pl.ds(wid * e + ch * rows_chunk,
                                           rows_chunk)], zrows_v)

            def ebody(i, carry):
                pred = (ch * rows_chunk + i) < nsplat
                out = []
                for s in range(nsl):
                    lanes_valid = 16 * s + iota < fdim
                    v = jnp.where(pred & lanes_valid,
                                  zrows_v[i, pl.ds(16 * s, 16)], 0.0)
                    out.append(carry[s] + v)
                return tuple(out)

            accs = lax.fori_loop(0, rows_chunk, ebody, tuple(accs))
        inv = 1.0 / jnp.maximum(nsplat, 1).astype(F32)
        for s in range(nsl):
            out_v[pl.ds(16 * s, 16)] = accs[s] * inv
        pltpu.sync_copy(out_v, out_hbm.at[wid])


def _head_body(x_ref, w1_ref, b1_ref, w2_ref, b2_ref, w3_ref, b3_ref,
               lg_ref, pred_ref):
    x = x_ref[...]
    h = jnp.maximum(lax.dot(x, w1_ref[...], precision=lax.Precision.HIGHEST,
                            preferred_element_type=F32) + b1_ref[...], 0.0)
    h = jnp.maximum(lax.dot(h, w2_ref[...], precision=lax.Precision.HIGHEST,
                            preferred_element_type=F32) + b2_ref[...], 0.0)
    lg = lax.dot(h, w3_ref[...], precision=lax.Precision.HIGHEST,
                 preferred_element_type=F32) + b3_ref[...]
    lg_ref[...] = lg
    ncls = lg.shape[1]
    col = lax.broadcasted_iota(I32, lg.shape, 1)
    mx = jnp.max(lg, axis=1, keepdims=True)
    pred_ref[...] = jnp.min(jnp.where(lg >= mx, col, ncls), axis=1,
                            keepdims=True)


def _tc_head(flat, w1, b1, w2, b2, w3, b3):
    b = flat.shape[0]
    ncls = w3.shape[1]
    return pl.pallas_call(
        _head_body,
        out_shape=(jax.ShapeDtypeStruct((b, ncls), F32),
                   jax.ShapeDtypeStruct((b, 1), I32)),
    )(flat, w1, b1, w2, b2, w3, b3)


def _build_wbig(wa, ws, fa, fs, p, out, dpad):
    kk = wa.shape[0] // (fa + fs)
    wa_r = wa.reshape(kk, fa + fs, out)
    ws_r = ws.reshape(kk, fa + fs, out)
    blocks = jnp.concatenate([wa_r, ws_r], axis=2)
    corr = -jnp.concatenate([wa_r[:, fa:, :].sum(0),
                             ws_r[:, fa:, :].sum(0)], axis=1)
    corr_full = jnp.zeros((fa + fs, 2 * out), F32).at[fa:].set(corr)
    wb = jnp.concatenate([blocks, corr_full[None]], axis=0)
    wb = jnp.pad(wb, ((0, 0), (0, p - (fa + fs)), (0, dpad - 2 * out)))
    perm = []
    for s in range(dpad // 32):
        for t in range(16):
            perm.extend((32 * s + t, 32 * s + 16 + t))
    wb = wb[:, :, jnp.array(perm, dtype=I32)]
    return wb


def kernel(space_features, all_features, neighbors_matrix, num_entries,
           params):
    b, e, fs0 = space_features.shape
    fa0 = all_features.shape[2]
    kk = neighbors_matrix.shape[2]
    nn = b * e
    nlayers = 6
    layer_out = [params['W%da' % l].shape[1] for l in range(nlayers)]

    nbr = neighbors_matrix.astype(I32)
    bofs = (jnp.arange(b, dtype=I32) * e)[:, None, None]
    gnbr = bofs + nbr
    idx_nbr = jnp.arange(kk, dtype=I32)[None, None, :] * nn + gnbr
    self_row = (kk * nn + bofs[..., 0]
                + jnp.arange(e, dtype=I32)[None, :])[:, :, None]
    idx = jnp.concatenate([idx_nbr, self_row], axis=2).reshape(-1)

    z = jnp.concatenate([all_features.reshape(nn, fa0),
                         space_features.reshape(nn, fs0)], axis=1)
    fa, fs = fa0, fs0
    for l in range(nlayers):
        out = layer_out[l]
        dpad = _ceil32(2 * out)
        p = z.shape[1]
        wb3 = _build_wbig(params['W%da' % l], params['W%ds' % l],
                          fa, fs, p, out, dpad)
        bias = jnp.pad(jnp.concatenate([params['b%da' % l],
                                        params['b%ds' % l]]),
                       (0, dpad - 2 * out))
        y3 = _tc_matmul_slots(z, wb3)
        ytab = y3.reshape(KP1 * nn, dpad)
        c = 128 if dpad <= 32 else 64
        z = _sc_gather_sum(ytab, idx, bias, nn, dpad, c)
        fa = fs = out

    n_rep = jnp.tile(num_entries.reshape(b, 1).astype(I32), (1, 16))
    flat = _sc_masked_mean(z, n_rep, b, e, z.shape[1], layer_out[-1])
    f1 = jnp.pad(params['fc1_w'], ((0, flat.shape[1] - layer_out[-1]),
                                   (0, 0)))
    logits, pred = _tc_head(flat, f1, params['fc1_b'][None],
                            params['fc2_w'], params['fc2_b'][None],
                            params['fc3_w'], params['fc3_b'][None])
    return logits, pred[:, 0]


# speedup vs baseline: 1.9399x; 1.1498x over previous
import functools

import jax
import jax.numpy as jnp
from jax import lax
from jax.experimental import pallas as pl
from jax.experimental.pallas import tpu as pltpu
from jax.experimental.pallas import tpu_sc as plsc

F32 = jnp.float32
BF16 = jnp.bfloat16
I32 = jnp.int32
NW = 32
KP1 = 17


def _ceil16(x):
    return (x + 15) // 16 * 16


def _ceil32(x):
    return (x + 31) // 32 * 32


def _mm17_body(*refs):
    x_ref, wh_ref, wl_ref = refs[:3]
    o_refs = refs[3:]
    x = x_ref[...]
    xh = x.astype(BF16)
    xl = (x - xh.astype(F32)).astype(BF16)
    for k, o_ref in enumerate(o_refs):
        acc = lax.dot(xh, wh_ref[k], preferred_element_type=F32)
        acc = acc + lax.dot(xh, wl_ref[k], preferred_element_type=F32)
        acc = acc + lax.dot(xl, wh_ref[k], preferred_element_type=F32)
        o_ref[...] = acc.astype(BF16)


def _tc_matmul_17(x, w3, bm=1024):
    m, p = x.shape
    nk, _, dpad = w3.shape
    w3h = w3.astype(BF16)
    w3l = (w3 - w3h.astype(F32)).astype(BF16)
    return pl.pallas_call(
        _mm17_body,
        grid=(m // bm,),
        in_specs=[pl.BlockSpec((bm, p), lambda i: (i, 0)),
                  pl.BlockSpec((nk, p, dpad), lambda i: (0, 0, 0)),
                  pl.BlockSpec((nk, p, dpad), lambda i: (0, 0, 0))],
        out_specs=[pl.BlockSpec((bm, dpad), lambda i: (i, 0))] * nk,
        out_shape=[jax.ShapeDtypeStruct((m, dpad), BF16)] * nk,
    )(x, w3h, w3l)


def _sc_gather_sum(ys, idx_t, bias, nn, dpad, c):
    npw = nn // NW
    nchunks = npw // c
    ngroups = dpad // 32
    mesh = plsc.VectorSubcoreMesh(core_axis_name="c", subcore_axis_name="s")

    @functools.partial(
        pl.kernel, mesh=mesh,
        out_type=jax.ShapeDtypeStruct((nn, dpad), F32),
        compiler_params=pltpu.CompilerParams(use_tc_tiling_on_sc=False,
                                             needs_layout_passes=False),
        scratch_types=[
            pltpu.VMEM((KP1, c), I32), pltpu.VMEM((KP1, c), I32),
            pltpu.VMEM((KP1, c, dpad), BF16),
            pltpu.VMEM((KP1, c, dpad), BF16),
            pltpu.VMEM((c, dpad), F32),
            pltpu.VMEM((dpad,), F32),
            pltpu.SemaphoreType.DMA, pltpu.SemaphoreType.DMA,
        ],
    )
    def k(*refs):
        y_hbms = refs[:KP1]
        idx_hbm, bias_hbm, out_hbm = refs[KP1:KP1 + 3]
        (idx_a, idx_b, rows_a, rows_b, out_v, bias_v, sem_a,
         sem_b) = refs[KP1 + 3:]
        wid = lax.axis_index("s") * 2 + lax.axis_index("c")
        base_node = wid * npw
        pltpu.sync_copy(bias_hbm, bias_v)
        idx_bufs, row_bufs, sems = (idx_a, idx_b), (rows_a, rows_b), \
            (sem_a, sem_b)

        def fire(g, bi):
            node0 = base_node + g * c
            pltpu.sync_copy(idx_hbm.at[:, pl.ds(node0, c)], idx_bufs[bi])
            for kk in range(KP1):
                pltpu.async_copy(y_hbms[kk].at[idx_bufs[bi].at[kk]],
                                 row_bufs[bi].at[kk], sems[bi])

        def drain(bi):
            for kk in range(KP1):
                pltpu.make_async_copy(y_hbms[kk].at[idx_bufs[bi].at[kk]],
                                      row_bufs[bi].at[kk], sems[bi]).wait()

        def process(g, bi):
            rows_v = row_bufs[bi]

            def acc_body(i, carry2):
                for grp in range(ngroups):
                    a32 = rows_v[0, i, pl.ds(32 * grp, 32)]
                    for kk in range(1, KP1):
                        a32 = a32 + rows_v[kk, i, pl.ds(32 * grp, 32)]
                    aa, ab = plsc.unpack(a32,
                                         format=plsc.PackFormat.INTERLEAVED)
                    sla = pl.ds(32 * grp, 16)
                    slb = pl.ds(32 * grp + 16, 16)
                    out_v[i, sla] = jnp.maximum(aa + bias_v[sla], 0.0)
                    out_v[i, slb] = jnp.maximum(ab + bias_v[slb], 0.0)
                return carry2

            lax.fori_loop(0, c, acc_body, 0)
            pltpu.sync_copy(out_v, out_hbm.at[pl.ds(base_node + g * c, c)])

        fire(0, 0)

        def group(to, carry):
            for bb in (0, 1):
                g = 2 * to + bb

                @pl.when(g + 1 < nchunks)
                def _():
                    fire(g + 1, 1 - bb)

                drain(bb)
                process(g, bb)
            return carry

        lax.fori_loop(0, nchunks // 2, group, 0)

    return k(*ys, idx_t, bias)


def _sc_masked_mean(z, n_arr, b, e, dpad, fdim):
    fpad = _ceil16(fdim)
    nsl = fpad // 16
    rows_chunk = 512
    nch = e // rows_chunk
    mesh = plsc.VectorSubcoreMesh(core_axis_name="c", subcore_axis_name="s")

    @functools.partial(
        pl.kernel, mesh=mesh,
        out_type=jax.ShapeDtypeStruct((b, fpad), F32),
        compiler_params=pltpu.CompilerParams(use_tc_tiling_on_sc=False),
        scratch_types=[
            pltpu.VMEM((rows_chunk, dpad), F32),
            pltpu.VMEM((16,), I32),
            pltpu.VMEM((fpad,), F32),
        ],
    )
    def k(z_hbm, n_hbm, out_hbm, zrows_v, n_v, out_v):
        wid = lax.axis_index("s") * 2 + lax.axis_index("c")

        @pl.when(wid < b)
        def _():
            _masked_mean_worker(z_hbm, n_hbm, out_hbm, zrows_v, n_v, out_v,
                                wid, e, dpad, fdim, nsl, rows_chunk, nch)

    return k(z, n_arr)


def _masked_mean_worker(z_hbm, n_hbm, out_hbm, zrows_v, n_v, out_v, wid, e,
                        dpad, fdim, nsl, rows_chunk, nch):
        pltpu.sync_copy(n_hbm.at[wid], n_v)
        nsplat = n_v[pl.ds(0, 16)]
        iota = lax.iota(I32, 16)
        accs = [jnp.zeros((16,), F32) for _ in range(nsl)]
        for ch in range(nch):
            pltpu.sync_copy(z_hbm.at[pl.ds(wid * e + ch * rows_chunk,
                                           rows_chunk)], zrows_v)

            def ebody(i, carry):
                pred = (ch * rows_chunk + i) < nsplat
                out = []
                for s in range(nsl):
                    lanes_valid = 16 * s + iota < fdim
                    v = jnp.where(pred & lanes_valid,
                                  zrows_v[i, pl.ds(16 * s, 16)], 0.0)
                    out.append(carry[s] + v)
                return tuple(out)

            accs = lax.fori_loop(0, rows_chunk, ebody, tuple(accs))
        inv = 1.0 / jnp.maximum(nsplat, 1).astype(F32)
        for s in range(nsl):
            out_v[pl.ds(16 * s, 16)] = accs[s] * inv
        pltpu.sync_copy(out_v, out_hbm.at[wid])


def _head_body(x_ref, w1_ref, b1_ref, w2_ref, b2_ref, w3_ref, b3_ref,
               lg_ref, pred_ref):
    x = x_ref[...]
    h = jnp.maximum(lax.dot(x, w1_ref[...], precision=lax.Precision.HIGHEST,
                            preferred_element_type=F32) + b1_ref[...], 0.0)
    h = jnp.maximum(lax.dot(h, w2_ref[...], precision=lax.Precision.HIGHEST,
                            preferred_element_type=F32) + b2_ref[...], 0.0)
    lg = lax.dot(h, w3_ref[...], precision=lax.Precision.HIGHEST,
                 preferred_element_type=F32) + b3_ref[...]
    lg_ref[...] = lg
    ncls = lg.shape[1]
    col = lax.broadcasted_iota(I32, lg.shape, 1)
    mx = jnp.max(lg, axis=1, keepdims=True)
    pred_ref[...] = jnp.min(jnp.where(lg >= mx, col, ncls), axis=1,
                            keepdims=True)


def _tc_head(flat, w1, b1, w2, b2, w3, b3):
    b = flat.shape[0]
    ncls = w3.shape[1]
    return pl.pallas_call(
        _head_body,
        out_shape=(jax.ShapeDtypeStruct((b, ncls), F32),
                   jax.ShapeDtypeStruct((b, 1), I32)),
    )(flat, w1, b1, w2, b2, w3, b3)


def _build_wbig(wa, ws, fa, fs, p, out, dpad):
    kk = wa.shape[0] // (fa + fs)
    wa_r = wa.reshape(kk, fa + fs, out)
    ws_r = ws.reshape(kk, fa + fs, out)
    blocks = jnp.concatenate([wa_r, ws_r], axis=2)
    corr = -jnp.concatenate([wa_r[:, fa:, :].sum(0),
                             ws_r[:, fa:, :].sum(0)], axis=1)
    corr_full = jnp.zeros((fa + fs, 2 * out), F32).at[fa:].set(corr)
    wb = jnp.concatenate([blocks, corr_full[None]], axis=0)
    wb = jnp.pad(wb, ((0, 0), (0, p - (fa + fs)), (0, dpad - 2 * out)))
    perm = []
    for s in range(dpad // 32):
        for t in range(16):
            perm.extend((32 * s + t, 32 * s + 16 + t))
    return wb[:, :, jnp.array(perm, dtype=I32)]


def kernel(space_features, all_features, neighbors_matrix, num_entries,
           params):
    b, e, fs0 = space_features.shape
    fa0 = all_features.shape[2]
    kk = neighbors_matrix.shape[2]
    nn = b * e
    nlayers = 6
    layer_out = [params['W%da' % l].shape[1] for l in range(nlayers)]

    nbr = neighbors_matrix.astype(I32)
    bofs = (jnp.arange(b, dtype=I32) * e)[:, None, None]
    gnbr = (bofs + nbr).reshape(nn, kk)
    idx_t = jnp.concatenate(
        [gnbr.T, jnp.arange(nn, dtype=I32)[None, :]], axis=0)

    z = jnp.concatenate([all_features.reshape(nn, fa0),
                         space_features.reshape(nn, fs0)], axis=1)
    fa, fs = fa0, fs0
    for l in range(nlayers):
        out = layer_out[l]
        dpad = _ceil32(2 * out)
        p = z.shape[1]
        wbig3 = _build_wbig(params['W%da' % l], params['W%ds' % l],
                            fa, fs, p, out, dpad)
        bias = jnp.pad(jnp.concatenate([params['b%da' % l],
                                        params['b%ds' % l]]),
                       (0, dpad - 2 * out))
        ys = _tc_matmul_17(z, wbig3)
        z = _sc_gather_sum(ys, idx_t, bias, nn, dpad, 64)
        fa = fs = out

    n_rep = jnp.tile(num_entries.reshape(b, 1).astype(I32), (1, 16))
    flat = _sc_masked_mean(z, n_rep, b, e, z.shape[1], layer_out[-1])
    f1 = jnp.pad(params['fc1_w'], ((0, flat.shape[1] - layer_out[-1]),
                                   (0, 0)))
    logits, pred = _tc_head(flat, f1, params['fc1_b'][None],
                            params['fc2_w'], params['fc2_b'][None],
                            params['fc3_w'], params['fc3_b'][None])
    return logits, pred[:, 0]


# speedup vs baseline: 2.5311x; 1.3048x over previous
import functools

import jax
import jax.numpy as jnp
from jax import lax
from jax.experimental import pallas as pl
from jax.experimental.pallas import tpu as pltpu
from jax.experimental.pallas import tpu_sc as plsc

F32 = jnp.float32
BF16 = jnp.bfloat16
I32 = jnp.int32
NW = 32
KP1 = 17


def _ceil16(x):
    return (x + 15) // 16 * 16


def _mm_body(x_ref, wh_ref, wl_ref, o_ref):
    x = x_ref[...]
    xh = x.astype(BF16)
    xl = (x - xh.astype(F32)).astype(BF16)
    acc = lax.dot(xh, wh_ref[...], preferred_element_type=F32)
    acc = acc + lax.dot(xh, wl_ref[...], preferred_element_type=F32)
    acc = acc + lax.dot(xl, wh_ref[...], preferred_element_type=F32)
    o_ref[...] = acc


def _tc_matmul(x, w, bm=512):
    m, p = x.shape
    n = w.shape[1]
    wh = w.astype(BF16)
    wl = (w - wh.astype(F32)).astype(BF16)
    return pl.pallas_call(
        _mm_body,
        grid=(m // bm,),
        in_specs=[pl.BlockSpec((bm, p), lambda i: (i, 0)),
                  pl.BlockSpec((p, n), lambda i: (0, 0)),
                  pl.BlockSpec((p, n), lambda i: (0, 0))],
        out_specs=pl.BlockSpec((bm, n), lambda i: (i, 0)),
        out_shape=jax.ShapeDtypeStruct((m, n), F32),
    )(x, wh, wl)


def _seg_list(total):
    segs, off = [], 0
    while off < total:
        seg = min(128, total - off)
        segs.append((off, seg))
        off += seg
    return segs


def _sc_gather_sum(ytab, idx, bias, nn, dpad, c):
    npw = nn // NW
    nchunks = npw // c
    segs = _seg_list(c * KP1)
    nslice = dpad // 16
    mesh = plsc.VectorSubcoreMesh(core_axis_name="c", subcore_axis_name="s")

    @functools.partial(
        pl.kernel, mesh=mesh,
        out_type=jax.ShapeDtypeStruct((nn, dpad), F32),
        compiler_params=pltpu.CompilerParams(use_tc_tiling_on_sc=False),
        scratch_types=[
            pltpu.VMEM((c * KP1,), I32), pltpu.VMEM((c * KP1,), I32),
            pltpu.VMEM((c * KP1, dpad), F32),
            pltpu.VMEM((c * KP1, dpad), F32),
            pltpu.VMEM((c, dpad), F32),
            pltpu.VMEM((dpad,), F32),
            pltpu.SemaphoreType.DMA, pltpu.SemaphoreType.DMA,
        ],
    )
    def k(y_hbm, idx_hbm, bias_hbm, out_hbm, idx_a, idx_b, rows_a, rows_b,
          out_v, bias_v, sem_a, sem_b):
        wid = lax.axis_index("s") * 2 + lax.axis_index("c")
        base_node = wid * npw
        pltpu.sync_copy(bias_hbm, bias_v)
        idx_bufs, row_bufs, sems = (idx_a, idx_b), (rows_a, rows_b), \
            (sem_a, sem_b)

        def fire(g, bi):
            node0 = base_node + g * c
            pltpu.sync_copy(idx_hbm.at[pl.ds(node0 * KP1, c * KP1)],
                            idx_bufs[bi])
            for off, seg in segs:
                pltpu.async_copy(y_hbm.at[idx_bufs[bi].at[pl.ds(off, seg)]],
                                 row_bufs[bi].at[pl.ds(off, seg)], sems[bi])

        def drain(bi):
            for off, seg in segs:
                pltpu.make_async_copy(
                    y_hbm.at[idx_bufs[bi].at[pl.ds(off, seg)]],
                    row_bufs[bi].at[pl.ds(off, seg)], sems[bi]).wait()

        def process(g, bi):
            rows_v = row_bufs[bi]

            def acc_body(i, carry2):
                r0 = i * KP1
                for sgrp in range(nslice):
                    sl = pl.ds(16 * sgrp, 16)
                    a = rows_v[r0, sl]
                    for kk in range(1, KP1):
                        a = a + rows_v[r0 + kk, sl]
                    out_v[i, sl] = jnp.maximum(a + bias_v[sl], 0.0)
                return carry2

            lax.fori_loop(0, c, acc_body, 0)
            pltpu.sync_copy(out_v, out_hbm.at[pl.ds(base_node + g * c, c)])

        fire(0, 0)

        def group(to, carry):
            for bb in (0, 1):
                g = 2 * to + bb

                @pl.when(g + 1 < nchunks)
                def _():
                    fire(g + 1, 1 - bb)

                drain(bb)
                process(g, bb)
            return carry

        lax.fori_loop(0, nchunks // 2, group, 0)

    return k(ytab, idx, bias)


def _sc_masked_mean(z, n_arr, b, e, dpad, fdim):
    fpad = _ceil16(fdim)
    nsl = fpad // 16
    rows_chunk = 512
    nch = e // rows_chunk
    mesh = plsc.VectorSubcoreMesh(core_axis_name="c", subcore_axis_name="s")

    @functools.partial(
        pl.kernel, mesh=mesh,
        out_type=jax.ShapeDtypeStruct((b, fpad), F32),
        compiler_params=pltpu.CompilerParams(use_tc_tiling_on_sc=False),
        scratch_types=[
            pltpu.VMEM((rows_chunk, dpad), F32),
            pltpu.VMEM((16,), I32),
            pltpu.VMEM((fpad,), F32),
        ],
    )
    def k(z_hbm, n_hbm, out_hbm, zrows_v, n_v, out_v):
        wid = lax.axis_index("s") * 2 + lax.axis_index("c")

        @pl.when(wid < b)
        def _():
            pltpu.sync_copy(n_hbm.at[wid], n_v)
            nsplat = n_v[pl.ds(0, 16)]
            iota = lax.iota(I32, 16)
            accs = [jnp.zeros((16,), F32) for _ in range(nsl)]
            for ch in range(nch):
                pltpu.sync_copy(z_hbm.at[pl.ds(wid * e + ch * rows_chunk,
                                               rows_chunk)], zrows_v)

                def ebody(i, carry):
                    pred = (ch * rows_chunk + i) < nsplat
                    outv = []
                    for sg in range(nsl):
                        lanes_valid = 16 * sg + iota < fdim
                        v = jnp.where(pred & lanes_valid,
                                      zrows_v[i, pl.ds(16 * sg, 16)], 0.0)
                        outv.append(carry[sg] + v)
                    return tuple(outv)

                accs = lax.fori_loop(0, rows_chunk, ebody, tuple(accs))
            inv = 1.0 / jnp.maximum(nsplat, 1).astype(F32)
            for sg in range(nsl):
                out_v[pl.ds(16 * sg, 16)] = accs[sg] * inv
            pltpu.sync_copy(out_v, out_hbm.at[wid])

    return k(z, n_arr)


def _head_body(x_ref, w1_ref, b1_ref, w2_ref, b2_ref, w3_ref, b3_ref,
               lg_ref, pred_ref):
    x = x_ref[...]
    h = jnp.maximum(lax.dot(x, w1_ref[...], precision=lax.Precision.HIGHEST,
                            preferred_element_type=F32) + b1_ref[...], 0.0)
    h = jnp.maximum(lax.dot(h, w2_ref[...], precision=lax.Precision.HIGHEST,
                            preferred_element_type=F32) + b2_ref[...], 0.0)
    lg = lax.dot(h, w3_ref[...], precision=lax.Precision.HIGHEST,
                 preferred_element_type=F32) + b3_ref[...]
    lg_ref[...] = lg
    ncls = lg.shape[1]
    col = lax.broadcasted_iota(I32, lg.shape, 1)
    mx = jnp.max(lg, axis=1, keepdims=True)
    pred_ref[...] = jnp.min(jnp.where(lg >= mx, col, ncls), axis=1,
                            keepdims=True)


def _tc_head(flat, w1, b1, w2, b2, w3, b3):
    b = flat.shape[0]
    ncls = w3.shape[1]
    return pl.pallas_call(
        _head_body,
        out_shape=(jax.ShapeDtypeStruct((b, ncls), F32),
                   jax.ShapeDtypeStruct((b, 1), I32)),
    )(flat, w1, b1, w2, b2, w3, b3)


def _build_wbig(wa, ws, fa, fs, p, out, dpad):
    kk = wa.shape[0] // (fa + fs)
    wa_r = wa.reshape(kk, fa + fs, out)
    ws_r = ws.reshape(kk, fa + fs, out)
    blocks = jnp.concatenate([wa_r, ws_r], axis=2)
    corr = -jnp.concatenate([wa_r[:, fa:, :].sum(0),
                             ws_r[:, fa:, :].sum(0)], axis=1)
    corr_full = jnp.zeros((fa + fs, 2 * out), F32).at[fa:].set(corr)
    wb = jnp.concatenate([blocks, corr_full[None]], axis=0)
    wb = jnp.pad(wb, ((0, 0), (0, p - (fa + fs)), (0, dpad - 2 * out)))
    return wb.transpose(1, 0, 2).reshape(p, KP1 * dpad)


def _chunk_nodes(dpad):
    for c in (64, 32, 16):
        if c * KP1 * dpad * 4 <= 220_000:
            return c
    return 16


def kernel(space_features, all_features, neighbors_matrix, num_entries,
           params):
    b, e, fs0 = space_features.shape
    fa0 = all_features.shape[2]
    kk = neighbors_matrix.shape[2]
    nn = b * e
    nlayers = 6
    layer_out = [params['W%da' % l].shape[1] for l in range(nlayers)]

    nbr = neighbors_matrix.astype(I32)
    bofs = (jnp.arange(b, dtype=I32) * e)[:, None, None]
    idx_nbr = (bofs + nbr) * KP1 + jnp.arange(kk, dtype=I32)[None, None, :]
    self_row = (bofs[..., 0] + jnp.arange(e, dtype=I32)[None, :]) * KP1 + kk
    idx = jnp.concatenate([idx_nbr, self_row[:, :, None]],
                          axis=2).reshape(-1)

    z = jnp.concatenate([all_features.reshape(nn, fa0),
                         space_features.reshape(nn, fs0)], axis=1)
    fa, fs = fa0, fs0
    for l in range(nlayers):
        out = layer_out[l]
        dpad = _ceil16(2 * out)
        p = z.shape[1]
        wbig = _build_wbig(params['W%da' % l], params['W%ds' % l],
                           fa, fs, p, out, dpad)
        bias = jnp.pad(jnp.concatenate([params['b%da' % l],
                                        params['b%ds' % l]]),
                       (0, dpad - 2 * out))
        y = _tc_matmul(z, wbig)
        ytab = y.reshape(nn * KP1, dpad)
        z = _sc_gather_sum(ytab, idx, bias, nn, dpad, _chunk_nodes(dpad))
        fa = fs = out

    n_rep = jnp.tile(num_entries.reshape(b, 1).astype(I32), (1, 16))
    flat = _sc_masked_mean(z, n_rep, b, e, z.shape[1], layer_out[-1])
    f1 = jnp.pad(params['fc1_w'], ((0, flat.shape[1] - layer_out[-1]),
                                   (0, 0)))
    logits, pred = _tc_head(flat, f1, params['fc1_b'][None],
                            params['fc2_w'], params['fc2_b'][None],
                            params['fc3_w'], params['fc3_b'][None])
    return logits, pred[:, 0]


# speedup vs baseline: 2.5317x; 1.0003x over previous
import functools

import jax
import jax.numpy as jnp
from jax import lax
from jax.experimental import pallas as pl
from jax.experimental.pallas import tpu as pltpu
from jax.experimental.pallas import tpu_sc as plsc

F32 = jnp.float32
BF16 = jnp.bfloat16
I32 = jnp.int32
NW = 32
KP1 = 17


def _ceil16(x):
    return (x + 15) // 16 * 16


def _mm_body(x_ref, wh_ref, wl_ref, o_ref):
    x = x_ref[...]
    xh = x.astype(BF16)
    xl = (x - xh.astype(F32)).astype(BF16)
    acc = lax.dot(xh, wh_ref[...], preferred_element_type=F32)
    acc = acc + lax.dot(xh, wl_ref[...], preferred_element_type=F32)
    acc = acc + lax.dot(xl, wh_ref[...], preferred_element_type=F32)
    o_ref[...] = acc.reshape(o_ref.shape)


def _tc_matmul(x, w, bm=512):
    m, p = x.shape
    n = w.shape[1]
    dpad = n // KP1
    wh = w.astype(BF16)
    wl = (w - wh.astype(F32)).astype(BF16)
    return pl.pallas_call(
        _mm_body,
        grid=(m // bm,),
        in_specs=[pl.BlockSpec((bm, p), lambda i: (i, 0)),
                  pl.BlockSpec((p, n), lambda i: (0, 0)),
                  pl.BlockSpec((p, n), lambda i: (0, 0))],
        out_specs=pl.BlockSpec((bm * KP1, dpad), lambda i: (i, 0)),
        out_shape=jax.ShapeDtypeStruct((m * KP1, dpad), F32),
    )(x, wh, wl)


def _seg_list(total):
    segs, off = [], 0
    while off < total:
        seg = min(128, total - off)
        segs.append((off, seg))
        off += seg
    return segs


def _sc_gather_sum(ytab, idx, bias, nn, dpad, c):
    npw = nn // NW
    nchunks = npw // c
    segs = _seg_list(c * KP1)
    nslice = dpad // 16
    mesh = plsc.VectorSubcoreMesh(core_axis_name="c", subcore_axis_name="s")

    @functools.partial(
        pl.kernel, mesh=mesh,
        out_type=jax.ShapeDtypeStruct((nn, dpad), F32),
        compiler_params=pltpu.CompilerParams(use_tc_tiling_on_sc=False),
        scratch_types=[
            pltpu.VMEM((c * KP1,), I32), pltpu.VMEM((c * KP1,), I32),
            pltpu.VMEM((c * KP1, dpad), F32),
            pltpu.VMEM((c * KP1, dpad), F32),
            pltpu.VMEM((c, dpad), F32),
            pltpu.VMEM((dpad,), F32),
            pltpu.SemaphoreType.DMA, pltpu.SemaphoreType.DMA,
        ],
    )
    def k(y_hbm, idx_hbm, bias_hbm, out_hbm, idx_a, idx_b, rows_a, rows_b,
          out_v, bias_v, sem_a, sem_b):
        wid = lax.axis_index("s") * 2 + lax.axis_index("c")
        base_node = wid * npw
        pltpu.sync_copy(bias_hbm, bias_v)
        idx_bufs, row_bufs, sems = (idx_a, idx_b), (rows_a, rows_b), \
            (sem_a, sem_b)

        def fire(g, bi):
            node0 = base_node + g * c
            pltpu.sync_copy(idx_hbm.at[pl.ds(node0 * KP1, c * KP1)],
                            idx_bufs[bi])
            for off, seg in segs:
                pltpu.async_copy(y_hbm.at[idx_bufs[bi].at[pl.ds(off, seg)]],
                                 row_bufs[bi].at[pl.ds(off, seg)], sems[bi])

        def drain(bi):
            for off, seg in segs:
                pltpu.make_async_copy(
                    y_hbm.at[idx_bufs[bi].at[pl.ds(off, seg)]],
                    row_bufs[bi].at[pl.ds(off, seg)], sems[bi]).wait()

        def process(g, bi):
            rows_v = row_bufs[bi]

            def acc_body(i, carry2):
                r0 = i * KP1
                for sgrp in range(nslice):
                    sl = pl.ds(16 * sgrp, 16)
                    a = rows_v[r0, sl]
                    for kk in range(1, KP1):
                        a = a + rows_v[r0 + kk, sl]
                    out_v[i, sl] = jnp.maximum(a + bias_v[sl], 0.0)
                return carry2

            lax.fori_loop(0, c, acc_body, 0)
            pltpu.sync_copy(out_v, out_hbm.at[pl.ds(base_node + g * c, c)])

        fire(0, 0)

        def group(to, carry):
            for bb in (0, 1):
                g = 2 * to + bb

                @pl.when(g + 1 < nchunks)
                def _():
                    fire(g + 1, 1 - bb)

                drain(bb)
                process(g, bb)
            return carry

        lax.fori_loop(0, nchunks // 2, group, 0)

    return k(ytab, idx, bias)


def _sc_masked_mean(z, n_arr, b, e, dpad, fdim):
    fpad = _ceil16(fdim)
    nsl = fpad // 16
    rows_chunk = 512
    nch = e // rows_chunk
    mesh = plsc.VectorSubcoreMesh(core_axis_name="c", subcore_axis_name="s")

    @functools.partial(
        pl.kernel, mesh=mesh,
        out_type=jax.ShapeDtypeStruct((b, fpad), F32),
        compiler_params=pltpu.CompilerParams(use_tc_tiling_on_sc=False),
        scratch_types=[
            pltpu.VMEM((rows_chunk, dpad), F32),
            pltpu.VMEM((16,), I32),
            pltpu.VMEM((fpad,), F32),
        ],
    )
    def k(z_hbm, n_hbm, out_hbm, zrows_v, n_v, out_v):
        wid = lax.axis_index("s") * 2 + lax.axis_index("c")

        @pl.when(wid < b)
        def _():
            pltpu.sync_copy(n_hbm.at[wid], n_v)
            nsplat = n_v[pl.ds(0, 16)]
            iota = lax.iota(I32, 16)
            accs = [jnp.zeros((16,), F32) for _ in range(nsl)]
            for ch in range(nch):
                pltpu.sync_copy(z_hbm.at[pl.ds(wid * e + ch * rows_chunk,
                                               rows_chunk)], zrows_v)

                def ebody(i, carry):
                    pred = (ch * rows_chunk + i) < nsplat
                    outv = []
                    for sg in range(nsl):
                        lanes_valid = 16 * sg + iota < fdim
                        v = jnp.where(pred & lanes_valid,
                                      zrows_v[i, pl.ds(16 * sg, 16)], 0.0)
                        outv.append(carry[sg] + v)
                    return tuple(outv)

                accs = lax.fori_loop(0, rows_chunk, ebody, tuple(accs))
            inv = 1.0 / jnp.maximum(nsplat, 1).astype(F32)
            for sg in range(nsl):
                out_v[pl.ds(16 * sg, 16)] = accs[sg] * inv
            pltpu.sync_copy(out_v, out_hbm.at[wid])

    return k(z, n_arr)


def _head_body(x_ref, w1_ref, b1_ref, w2_ref, b2_ref, w3_ref, b3_ref,
               lg_ref, pred_ref):
    x = x_ref[...]
    h = jnp.maximum(lax.dot(x, w1_ref[...], precision=lax.Precision.HIGHEST,
                            preferred_element_type=F32) + b1_ref[...], 0.0)
    h = jnp.maximum(lax.dot(h, w2_ref[...], precision=lax.Precision.HIGHEST,
                            preferred_element_type=F32) + b2_ref[...], 0.0)
    lg = lax.dot(h, w3_ref[...], precision=lax.Precision.HIGHEST,
                 preferred_element_type=F32) + b3_ref[...]
    lg_ref[...] = lg
    ncls = lg.shape[1]
    col = lax.broadcasted_iota(I32, lg.shape, 1)
    mx = jnp.max(lg, axis=1, keepdims=True)
    pred_ref[...] = jnp.min(jnp.where(lg >= mx, col, ncls), axis=1,
                            keepdims=True)


def _tc_head(flat, w1, b1, w2, b2, w3, b3):
    b = flat.shape[0]
    ncls = w3.shape[1]
    return pl.pallas_call(
        _head_body,
        out_shape=(jax.ShapeDtypeStruct((b, ncls), F32),
                   jax.ShapeDtypeStruct((b, 1), I32)),
    )(flat, w1, b1, w2, b2, w3, b3)


def _build_wbig(wa, ws, fa, fs, p, out, dpad):
    kk = wa.shape[0] // (fa + fs)
    wa_r = wa.reshape(kk, fa + fs, out)
    ws_r = ws.reshape(kk, fa + fs, out)
    blocks = jnp.concatenate([wa_r, ws_r], axis=2)
    corr = -jnp.concatenate([wa_r[:, fa:, :].sum(0),
                             ws_r[:, fa:, :].sum(0)], axis=1)
    corr_full = jnp.zeros((fa + fs, 2 * out), F32).at[fa:].set(corr)
    wb = jnp.concatenate([blocks, corr_full[None]], axis=0)
    wb = jnp.pad(wb, ((0, 0), (0, p - (fa + fs)), (0, dpad - 2 * out)))
    return wb.transpose(1, 0, 2).reshape(p, KP1 * dpad)


def _chunk_nodes(dpad):
    for c in (64, 32, 16):
        if c * KP1 * dpad * 4 <= 150_000:
            return c
    return 16


def kernel(space_features, all_features, neighbors_matrix, num_entries,
           params):
    b, e, fs0 = space_features.shape
    fa0 = all_features.shape[2]
    kk = neighbors_matrix.shape[2]
    nn = b * e
    nlayers = 6
    layer_out = [params['W%da' % l].shape[1] for l in range(nlayers)]

    nbr = neighbors_matrix.astype(I32)
    bofs = (jnp.arange(b, dtype=I32) * e)[:, None, None]
    idx_nbr = (bofs + nbr) * KP1 + jnp.arange(kk, dtype=I32)[None, None, :]
    self_row = (bofs[..., 0] + jnp.arange(e, dtype=I32)[None, :]) * KP1 + kk
    idx = jnp.concatenate([idx_nbr, self_row[:, :, None]],
                          axis=2).reshape(-1)

    z = jnp.concatenate([all_features.reshape(nn, fa0),
                         space_features.reshape(nn, fs0)], axis=1)
    fa, fs = fa0, fs0
    for l in range(nlayers):
        out = layer_out[l]
        dpad = 128
        p = z.shape[1]
        wbig = _build_wbig(params['W%da' % l], params['W%ds' % l],
                           fa, fs, p, out, dpad)
        bias = jnp.pad(jnp.concatenate([params['b%da' % l],
                                        params['b%ds' % l]]),
                       (0, dpad - 2 * out))
        ytab = _tc_matmul(z, wbig)
        z = _sc_gather_sum(ytab, idx, bias, nn, dpad, _chunk_nodes(dpad))
        fa = fs = out

    n_rep = jnp.tile(num_entries.reshape(b, 1).astype(I32), (1, 16))
    flat = _sc_masked_mean(z, n_rep, b, e, z.shape[1], layer_out[-1])
    f1 = jnp.pad(params['fc1_w'], ((0, flat.shape[1] - layer_out[-1]),
                                   (0, 0)))
    logits, pred = _tc_head(flat, f1, params['fc1_b'][None],
                            params['fc2_w'], params['fc2_b'][None],
                            params['fc3_w'], params['fc3_b'][None])
    return logits, pred[:, 0]


# speedup vs baseline: 2.6413x; 1.0433x over previous
import functools

import jax
import jax.numpy as jnp
from jax import lax
from jax.experimental import pallas as pl
from jax.experimental.pallas import tpu as pltpu
from jax.experimental.pallas import tpu_sc as plsc

F32 = jnp.float32
BF16 = jnp.bfloat16
I32 = jnp.int32
NW = 32
KP1 = 17


def _ceil16(x):
    return (x + 15) // 16 * 16


def _mm_body(x_ref, w_ref, o_ref):
    o_ref[...] = lax.dot(x_ref[...], w_ref[...],
                         precision=lax.Precision.HIGHEST,
                         preferred_element_type=F32)


def _tc_matmul(x, w, bm=512):
    m, p = x.shape
    n = w.shape[1]
    return pl.pallas_call(
        _mm_body,
        grid=(m // bm,),
        in_specs=[pl.BlockSpec((bm, p), lambda i: (i, 0)),
                  pl.BlockSpec((p, n), lambda i: (0, 0))],
        out_specs=pl.BlockSpec((bm, n), lambda i: (i, 0)),
        out_shape=jax.ShapeDtypeStruct((m, n), F32),
    )(x, w)


def _seg_list(total):
    segs, off = [], 0
    while off < total:
        seg = min(128, total - off)
        segs.append((off, seg))
        off += seg
    return segs


def _sc_gather_sum(ytab, idx, bias, nn, dpad, c):
    npw = nn // NW
    nchunks = npw // c
    segs = _seg_list(c * KP1)
    nslice = dpad // 16
    mesh = plsc.VectorSubcoreMesh(core_axis_name="c", subcore_axis_name="s")

    @functools.partial(
        pl.kernel, mesh=mesh,
        out_type=jax.ShapeDtypeStruct((nn, dpad), F32),
        compiler_params=pltpu.CompilerParams(use_tc_tiling_on_sc=False),
        scratch_types=[
            pltpu.VMEM((c * KP1,), I32), pltpu.VMEM((c * KP1,), I32),
            pltpu.VMEM((c * KP1, dpad), F32),
            pltpu.VMEM((c * KP1, dpad), F32),
            pltpu.VMEM((c, dpad), F32),
            pltpu.VMEM((dpad,), F32),
            pltpu.SemaphoreType.DMA, pltpu.SemaphoreType.DMA,
        ],
    )
    def k(y_hbm, idx_hbm, bias_hbm, out_hbm, idx_a, idx_b, rows_a, rows_b,
          out_v, bias_v, sem_a, sem_b):
        wid = lax.axis_index("s") * 2 + lax.axis_index("c")
        base_node = wid * npw
        pltpu.sync_copy(bias_hbm, bias_v)
        idx_bufs, row_bufs, sems = (idx_a, idx_b), (rows_a, rows_b), \
            (sem_a, sem_b)

        def fire(g, bi):
            node0 = base_node + g * c
            pltpu.sync_copy(idx_hbm.at[pl.ds(node0 * KP1, c * KP1)],
                            idx_bufs[bi])
            for off, seg in segs:
                pltpu.async_copy(y_hbm.at[idx_bufs[bi].at[pl.ds(off, seg)]],
                                 row_bufs[bi].at[pl.ds(off, seg)], sems[bi])

        def drain(bi):
            for off, seg in segs:
                pltpu.make_async_copy(
                    y_hbm.at[idx_bufs[bi].at[pl.ds(off, seg)]],
                    row_bufs[bi].at[pl.ds(off, seg)], sems[bi]).wait()

        def process(g, bi):
            rows_v = row_bufs[bi]

            def acc_body(i, carry2):
                r0 = i * KP1
                for sgrp in range(nslice):
                    sl = pl.ds(16 * sgrp, 16)
                    a = rows_v[r0, sl]
                    for kk in range(1, KP1):
                        a = a + rows_v[r0 + kk, sl]
                    out_v[i, sl] = jnp.maximum(a + bias_v[sl], 0.0)
                return carry2

            lax.fori_loop(0, c, acc_body, 0)
            pltpu.sync_copy(out_v, out_hbm.at[pl.ds(base_node + g * c, c)])

        fire(0, 0)

        def group(to, carry):
            for bb in (0, 1):
                g = 2 * to + bb

                @pl.when(g + 1 < nchunks)
                def _():
                    fire(g + 1, 1 - bb)

                drain(bb)
                process(g, bb)
            return carry

        lax.fori_loop(0, nchunks // 2, group, 0)

    return k(ytab, idx, bias)


def _sc_masked_mean(z, n_arr, b, e, dpad, fdim):
    fpad = _ceil16(fdim)
    nsl = fpad // 16
    rows_chunk = 512
    nch = e // rows_chunk
    mesh = plsc.VectorSubcoreMesh(core_axis_name="c", subcore_axis_name="s")

    @functools.partial(
        pl.kernel, mesh=mesh,
        out_type=jax.ShapeDtypeStruct((b, fpad), F32),
        compiler_params=pltpu.CompilerParams(use_tc_tiling_on_sc=False),
        scratch_types=[
            pltpu.VMEM((rows_chunk, dpad), F32),
            pltpu.VMEM((16,), I32),
            pltpu.VMEM((fpad,), F32),
        ],
    )
    def k(z_hbm, n_hbm, out_hbm, zrows_v, n_v, out_v):
        wid = lax.axis_index("s") * 2 + lax.axis_index("c")

        @pl.when(wid < b)
        def _():
            pltpu.sync_copy(n_hbm.at[wid], n_v)
            nsplat = n_v[pl.ds(0, 16)]
            iota = lax.iota(I32, 16)
            accs = [jnp.zeros((16,), F32) for _ in range(nsl)]
            for ch in range(nch):
                pltpu.sync_copy(z_hbm.at[pl.ds(wid * e + ch * rows_chunk,
                                               rows_chunk)], zrows_v)

                def ebody(i, carry):
                    pred = (ch * rows_chunk + i) < nsplat
                    outv = []
                    for sg in range(nsl):
                        lanes_valid = 16 * sg + iota < fdim
                        v = jnp.where(pred & lanes_valid,
                                      zrows_v[i, pl.ds(16 * sg, 16)], 0.0)
                        outv.append(carry[sg] + v)
                    return tuple(outv)

                accs = lax.fori_loop(0, rows_chunk, ebody, tuple(accs))
            inv = 1.0 / jnp.maximum(nsplat, 1).astype(F32)
            for sg in range(nsl):
                out_v[pl.ds(16 * sg, 16)] = accs[sg] * inv
            pltpu.sync_copy(out_v, out_hbm.at[wid])

    return k(z, n_arr)


def _head_body(x_ref, w1_ref, b1_ref, w2_ref, b2_ref, w3_ref, b3_ref,
               lg_ref, pred_ref):
    x = x_ref[...]
    h = jnp.maximum(lax.dot(x, w1_ref[...], precision=lax.Precision.HIGHEST,
                            preferred_element_type=F32) + b1_ref[...], 0.0)
    h = jnp.maximum(lax.dot(h, w2_ref[...], precision=lax.Precision.HIGHEST,
                            preferred_element_type=F32) + b2_ref[...], 0.0)
    lg = lax.dot(h, w3_ref[...], precision=lax.Precision.HIGHEST,
                 preferred_element_type=F32) + b3_ref[...]
    lg_ref[...] = lg
    ncls = lg.shape[1]
    col = lax.broadcasted_iota(I32, lg.shape, 1)
    mx = jnp.max(lg, axis=1, keepdims=True)
    pred_ref[...] = jnp.min(jnp.where(lg >= mx, col, ncls), axis=1,
                            keepdims=True)


def _tc_head(flat, w1, b1, w2, b2, w3, b3):
    b = flat.shape[0]
    ncls = w3.shape[1]
    return pl.pallas_call(
        _head_body,
        out_shape=(jax.ShapeDtypeStruct((b, ncls), F32),
                   jax.ShapeDtypeStruct((b, 1), I32)),
    )(flat, w1, b1, w2, b2, w3, b3)


def _build_wbig(wa, ws, fa, fs, p, out, dpad):
    kk = wa.shape[0] // (fa + fs)
    wa_r = wa.reshape(kk, fa + fs, out)
    ws_r = ws.reshape(kk, fa + fs, out)
    blocks = jnp.concatenate([wa_r, ws_r], axis=2)
    corr = -jnp.concatenate([wa_r[:, fa:, :].sum(0),
                             ws_r[:, fa:, :].sum(0)], axis=1)
    corr_full = jnp.zeros((fa + fs, 2 * out), F32).at[fa:].set(corr)
    wb = jnp.concatenate([blocks, corr_full[None]], axis=0)
    wb = jnp.pad(wb, ((0, 0), (0, p - (fa + fs)), (0, dpad - 2 * out)))
    return wb.transpose(1, 0, 2).reshape(p, KP1 * dpad)


def _chunk_nodes(dpad):
    for c in (64, 32, 16):
        if c * KP1 * dpad * 4 <= 220_000:
            return c
    return 16


def kernel(space_features, all_features, neighbors_matrix, num_entries,
           params):
    b, e, fs0 = space_features.shape
    fa0 = all_features.shape[2]
    kk = neighbors_matrix.shape[2]
    nn = b * e
    nlayers = 6
    layer_out = [params['W%da' % l].shape[1] for l in range(nlayers)]

    nbr = neighbors_matrix.astype(I32)
    bofs = (jnp.arange(b, dtype=I32) * e)[:, None, None]
    idx_nbr = (bofs + nbr) * KP1 + jnp.arange(kk, dtype=I32)[None, None, :]
    self_row = (bofs[..., 0] + jnp.arange(e, dtype=I32)[None, :]) * KP1 + kk
    idx = jnp.concatenate([idx_nbr, self_row[:, :, None]],
                          axis=2).reshape(-1)

    z = jnp.concatenate([all_features.reshape(nn, fa0),
                         space_features.reshape(nn, fs0)], axis=1)
    fa, fs = fa0, fs0
    for l in range(nlayers):
        out = layer_out[l]
        dpad = _ceil16(2 * out)
        p = z.shape[1]
        wbig = _build_wbig(params['W%da' % l], params['W%ds' % l],
                           fa, fs, p, out, dpad)
        bias = jnp.pad(jnp.concatenate([params['b%da' % l],
                                        params['b%ds' % l]]),
                       (0, dpad - 2 * out))
        y = _tc_matmul(z, wbig)
        ytab = y.reshape(nn * KP1, dpad)
        z = _sc_gather_sum(ytab, idx, bias, nn, dpad, _chunk_nodes(dpad))
        fa = fs = out

    n_rep = jnp.tile(num_entries.reshape(b, 1).astype(I32), (1, 16))
    flat = _sc_masked_mean(z, n_rep, b, e, z.shape[1], layer_out[-1])
    f1 = jnp.pad(params['fc1_w'], ((0, flat.shape[1] - layer_out[-1]),
                                   (0, 0)))
    logits, pred = _tc_head(flat, f1, params['fc1_b'][None],
                            params['fc2_w'], params['fc2_b'][None],
                            params['fc3_w'], params['fc3_b'][None])
    return logits, pred[:, 0]
